# baseline (device time: 6642 ns/iter reference)
import jax
import jax.numpy as jnp
from jax import lax
from jax.experimental import pallas as pl
from jax.experimental.pallas import tpu as pltpu

N_CHUNKS = 4


def kernel(x):
    m, n = x.shape
    w = n // N_CHUNKS

    def body(x_ref, out_ref, comm_ref, send_sems, recv_sems):
        j = pl.program_id(0)
        my_x = lax.axis_index("x")
        my_y = lax.axis_index("y")
        peer = (1 - my_x, my_y)

        barrier_sem = pltpu.get_barrier_semaphore()

        @pl.when(j == 0)
        def _():
            pl.semaphore_signal(
                barrier_sem, inc=1, device_id=peer,
                device_id_type=pl.DeviceIdType.MESH,
            )

        comm_ref[0, j, :, :] = jnp.max(x_ref[:, :], axis=0, keepdims=True)

        @pl.when(j == 0)
        def _():
            pl.semaphore_wait(barrier_sem, 1)

        rdma = pltpu.make_async_remote_copy(
            src_ref=comm_ref.at[0, j],
            dst_ref=comm_ref.at[1, j],
            send_sem=send_sems.at[j],
            recv_sem=recv_sems.at[j],
            device_id=peer,
            device_id_type=pl.DeviceIdType.MESH,
        )
        rdma.start()

        @pl.when(j == N_CHUNKS - 1)
        def _():
            for jj in range(N_CHUNKS):
                d = pltpu.make_async_remote_copy(
                    src_ref=comm_ref.at[0, jj],
                    dst_ref=comm_ref.at[1, jj],
                    send_sem=send_sems.at[jj],
                    recv_sem=recv_sems.at[jj],
                    device_id=peer,
                    device_id_type=pl.DeviceIdType.MESH,
                )
                d.wait()
                out_ref[:, pl.ds(jj * w, w)] = jnp.maximum(
                    comm_ref[0, jj, :, :], comm_ref[1, jj, :, :]
                )

    return pl.pallas_call(
        body,
        grid=(N_CHUNKS,),
        out_shape=jax.ShapeDtypeStruct((1, n), x.dtype),
        in_specs=[
            pl.BlockSpec((m, w), lambda j: (0, j), memory_space=pltpu.VMEM)
        ],
        out_specs=pl.BlockSpec((1, n), lambda j: (0, 0), memory_space=pltpu.VMEM),
        scratch_shapes=[
            pltpu.VMEM((2, N_CHUNKS, 1, w), x.dtype),
            pltpu.SemaphoreType.DMA((N_CHUNKS,)),
            pltpu.SemaphoreType.DMA((N_CHUNKS,)),
        ],
        compiler_params=pltpu.CompilerParams(collective_id=0),
    )(x)
